# baseline (speedup 1.0000x reference)
import functools

import jax
import jax.numpy as jnp
from jax import lax
from jax.experimental import pallas as pl
from jax.experimental.pallas import tpu as pltpu
from jax.experimental.pallas import tpu_sc as plsc

N = 10000
D = 256
H = 128
E = 160000
SUBC = 16
EPT_M = E // SUBC
CHUNK = 80
NPAIR = EPT_M // CHUNK
NCH_M = E // CHUNK
NSLOT = 3
ROWS_PER_TILE = 624
ROWS_REM = N - SUBC * ROWS_PER_TILE
UNROLL = 16
VROWS = CHUNK * 16 // 128


def _sc_spmm(y_lo, y_hi, cols3, rows3, vals3, bb_lo, bb_hi):
    mesh = plsc.VectorSubcoreMesh(core_axis_name="c", subcore_axis_name="s")

    @functools.partial(
        pl.kernel,
        mesh=mesh,
        out_type=jax.ShapeDtypeStruct((N, D), jnp.float32),
        scratch_types=[
            pltpu.VMEM((NSLOT, CHUNK), jnp.int32),
            pltpu.VMEM((NSLOT, CHUNK), jnp.int32),
            pltpu.VMEM((NSLOT, CHUNK), jnp.int32),
            pltpu.VMEM((NSLOT, VROWS, 128), jnp.float32),
            pltpu.VMEM((NSLOT, CHUNK, H), jnp.float32),
            pltpu.VMEM_SHARED((N, H), jnp.float32),
            [pltpu.SemaphoreType.DMA] * NSLOT,
            [pltpu.SemaphoreType.DMA] * NSLOT,
            [pltpu.SemaphoreType.DMA] * NSLOT,
        ],
    )
    def k(ylo_hbm, yhi_hbm, c0_hbm, c1_hbm, c2_hbm, r0_hbm, r1_hbm, r2_hbm,
          v0_hbm, v1_hbm, v2_hbm, blo_hbm, bhi_hbm, out_hbm,
          colsb, rowsb, rows_s, vals_v, g_v, acc, sg, sv, ss):
        c = lax.axis_index("c")
        s = lax.axis_index("s")
        r0 = s * ROWS_PER_TILE
        e0 = s * EPT_M
        c0 = s * NPAIR
        cols_h = (c0_hbm, c1_hbm, c2_hbm)
        rows_h = (r0_hbm, r1_hbm, r2_hbm)
        vals_h = (v0_hbm, v1_hbm, v2_hbm)

        def init_acc(b_hbm):
            pltpu.sync_copy(b_hbm, acc.at[pl.ds(r0, ROWS_PER_TILE)])

            @pl.when(s == 0)
            def _():
                pltpu.sync_copy(b_hbm.at[pl.ds(0, ROWS_REM)],
                                acc.at[pl.ds(SUBC * ROWS_PER_TILE,
                                             ROWS_REM)])

        @pl.when(c == 0)
        def _():
            init_acc(blo_hbm)

        @pl.when(c == 1)
        def _():
            init_acc(bhi_hbm)

        plsc.subcore_barrier()

        def edge_pass(y_hbm):
            def start_meta(tt, p):
                off = e0 + tt * CHUNK
                pltpu.async_copy(cols_h[p].at[pl.ds(off, CHUNK)],
                                 colsb.at[p], sv[p])
                pltpu.async_copy(rows_h[p].at[pl.ds(off, CHUNK)],
                                 rowsb.at[p], sv[p])
                pltpu.async_copy(vals_h[p].at[c0 + tt], vals_v.at[p],
                                 sv[p])

            def wait_meta(p):
                pltpu.make_async_copy(cols_h[p].at[pl.ds(e0, CHUNK)],
                                      colsb.at[p], sv[p]).wait()
                pltpu.make_async_copy(rows_h[p].at[pl.ds(e0, CHUNK)],
                                      rowsb.at[p], sv[p]).wait()
                pltpu.make_async_copy(vals_h[p].at[c0], vals_v.at[p],
                                      sv[p]).wait()

            def start_gather(p):
                pltpu.async_copy(y_hbm.at[colsb.at[p]], g_v.at[p], sg[p])

            def wait_gather(p):
                pltpu.make_async_copy(y_hbm.at[colsb.at[p]], g_v.at[p],
                                      sg[p]).wait()

            def start_scat(p):
                pltpu.async_copy(g_v.at[p], acc.at[rows_s.at[p]], ss[p],
                                 add=True)

            def wait_scat(p):
                pltpu.make_async_copy(g_v.at[p], acc.at[rows_s.at[p]],
                                      ss[p]).wait()

            def scale(p):
                for i in range(CHUNK):
                    vv = vals_v[p, i // 8, pl.ds((i % 8) * 16, 16)]
                    for r in range(H // 16):
                        sl = pl.ds(r * 16, 16)
                        g_v[p, i, sl] = g_v[p, i, sl] * vv

            for p in range(NSLOT):
                start_meta(0, p)
            wait_meta(0)
            start_gather(0)

            def iter_body(t, carry):
                for u in range(NSLOT):
                    v = (u + 1) % NSLOT

                    def prefetch():
                        if u == NSLOT - 1:
                            wait_scat(v)
                        else:
                            @pl.when(t >= 1)
                            def _():
                                wait_scat(v)

                        wait_meta(v)
                        start_gather(v)

                    if u == NSLOT - 1:
                        @pl.when(t < NPAIR - 1)
                        def _():
                            prefetch()
                    else:
                        prefetch()

                    wait_gather(u)
                    scale(u)
                    for r in range(CHUNK // 16):
                        sl = pl.ds(r * 16, 16)
                        rows_s[u, sl] = rowsb[u, sl]
                    start_scat(u)

                    @pl.when(t < NPAIR - 1)
                    def _():
                        start_meta(t + 1, u)
                return carry

            lax.fori_loop(0, NPAIR, iter_body, 0)
            for p in range(NSLOT):
                wait_scat(p)

        @pl.when(c == 0)
        def _():
            edge_pass(ylo_hbm)

        @pl.when(c == 1)
        def _():
            edge_pass(yhi_hbm)

        plsc.subcore_barrier()

        def wb(col0):
            pltpu.sync_copy(
                acc.at[pl.ds(r0, ROWS_PER_TILE)],
                out_hbm.at[pl.ds(r0, ROWS_PER_TILE), pl.ds(col0, H)])

            @pl.when(s == 0)
            def _():
                tail = SUBC * ROWS_PER_TILE
                pltpu.sync_copy(
                    acc.at[pl.ds(tail, ROWS_REM)],
                    out_hbm.at[pl.ds(tail, ROWS_REM), pl.ds(col0, H)])

        @pl.when(c == 0)
        def _():
            wb(0)

        @pl.when(c == 1)
        def _():
            wb(H)

    return k(y_lo, y_hi, *cols3, *rows3, *vals3, bb_lo, bb_hi)


def _tc_matmul(x, w):
    BM = 1000

    def mm(x_ref, w_ref, ylo_ref, yhi_ref):
        y = jnp.dot(x_ref[...], w_ref[...],
                    preferred_element_type=jnp.float32)
        ylo_ref[...] = y[:, :H]
        yhi_ref[...] = y[:, H:]

    return pl.pallas_call(
        mm,
        grid=(N // BM,),
        in_specs=[
            pl.BlockSpec((BM, D), lambda m: (m, 0)),
            pl.BlockSpec((D, D), lambda m: (0, 0)),
        ],
        out_specs=(
            pl.BlockSpec((BM, H), lambda m: (m, 0)),
            pl.BlockSpec((BM, H), lambda m: (m, 0)),
        ),
        out_shape=(
            jax.ShapeDtypeStruct((N, H), jnp.float32),
            jax.ShapeDtypeStruct((N, H), jnp.float32),
        ),
    )(x, w)


@jax.jit
def kernel(x, F0_rows, F0_cols, F0_vals, F1_rows, F1_cols, F1_vals,
           F2_rows, F2_cols, F2_vals, W, b):
    y_lo, y_hi = _tc_matmul(x, W)
    vals3 = tuple(
        jnp.broadcast_to(v.reshape(NCH_M, CHUNK, 1),
                         (NCH_M, CHUNK, 16)).reshape(NCH_M, VROWS, 128)
        for v in (F0_vals, F1_vals, F2_vals))
    bb_lo = jnp.broadcast_to(b[None, :H], (ROWS_PER_TILE, H))
    bb_hi = jnp.broadcast_to(b[None, H:], (ROWS_PER_TILE, H))
    return _sc_spmm(y_lo, y_hi, (F0_cols, F1_cols, F2_cols),
                    (F0_rows, F1_rows, F2_rows), vals3, bb_lo, bb_hi)


# speedup vs baseline: 1.2256x; 1.2256x over previous
import functools

import jax
import jax.numpy as jnp
from jax import lax
from jax.experimental import pallas as pl
from jax.experimental.pallas import tpu as pltpu
from jax.experimental.pallas import tpu_sc as plsc

N = 10000
D = 256
H = 128
E = 160000
SUBC = 16
EPT_M = E // SUBC
CHUNK = 80
NPAIR = EPT_M // CHUNK
NCH_M = E // CHUNK
NSLOT = 3
ROWS_PER_TILE = 624
ROWS_REM = N - SUBC * ROWS_PER_TILE
UNROLL = 8
VROWS = CHUNK * 16 // 128


def _sc_spmm(y_lo, y_hi, cols3, rows3, vals3, bb_lo, bb_hi):
    mesh = plsc.VectorSubcoreMesh(core_axis_name="c", subcore_axis_name="s")

    @functools.partial(
        pl.kernel,
        mesh=mesh,
        out_type=jax.ShapeDtypeStruct((N, D), jnp.float32),
        scratch_types=[
            pltpu.VMEM((NSLOT, CHUNK), jnp.int32),
            pltpu.VMEM((NSLOT, CHUNK), jnp.int32),
            pltpu.VMEM((NSLOT, CHUNK), jnp.int32),
            pltpu.VMEM((NSLOT, VROWS, 128), jnp.float32),
            pltpu.VMEM((NSLOT, CHUNK, H), jnp.float32),
            pltpu.VMEM_SHARED((N, H), jnp.float32),
            [pltpu.SemaphoreType.DMA] * NSLOT,
            [pltpu.SemaphoreType.DMA] * NSLOT,
            [pltpu.SemaphoreType.DMA] * NSLOT,
        ],
    )
    def k(ylo_hbm, yhi_hbm, c0_hbm, c1_hbm, c2_hbm, r0_hbm, r1_hbm, r2_hbm,
          v0_hbm, v1_hbm, v2_hbm, blo_hbm, bhi_hbm, out_hbm,
          colsb, rowsb, rows_s, vals_v, g_v, acc, sg, sv, ss):
        c = lax.axis_index("c")
        s = lax.axis_index("s")
        r0 = s * ROWS_PER_TILE
        e0 = s * EPT_M
        c0 = s * NPAIR
        cols_h = (c0_hbm, c1_hbm, c2_hbm)
        rows_h = (r0_hbm, r1_hbm, r2_hbm)
        vals_h = (v0_hbm, v1_hbm, v2_hbm)

        def init_acc(b_hbm):
            pltpu.sync_copy(b_hbm, acc.at[pl.ds(r0, ROWS_PER_TILE)])

            @pl.when(s == 0)
            def _():
                pltpu.sync_copy(b_hbm.at[pl.ds(0, ROWS_REM)],
                                acc.at[pl.ds(SUBC * ROWS_PER_TILE,
                                             ROWS_REM)])

        @pl.when(c == 0)
        def _():
            init_acc(blo_hbm)

        @pl.when(c == 1)
        def _():
            init_acc(bhi_hbm)

        plsc.subcore_barrier()

        def edge_pass():
            def start_meta(tt, p):
                off = e0 + tt * CHUNK
                pltpu.async_copy(cols_h[p].at[pl.ds(off, CHUNK)],
                                 colsb.at[p], sv[p])
                pltpu.async_copy(rows_h[p].at[pl.ds(off, CHUNK)],
                                 rowsb.at[p], sv[p])
                pltpu.async_copy(vals_h[p].at[c0 + tt], vals_v.at[p],
                                 sv[p])

            def wait_meta(p):
                pltpu.make_async_copy(cols_h[p].at[pl.ds(e0, CHUNK)],
                                      colsb.at[p], sv[p]).wait()
                pltpu.make_async_copy(rows_h[p].at[pl.ds(e0, CHUNK)],
                                      rowsb.at[p], sv[p]).wait()
                pltpu.make_async_copy(vals_h[p].at[c0], vals_v.at[p],
                                      sv[p]).wait()

            def start_gather(p):
                @pl.when(c == 0)
                def _():
                    pltpu.async_copy(ylo_hbm.at[colsb.at[p]], g_v.at[p],
                                     sg[p])

                @pl.when(c == 1)
                def _():
                    pltpu.async_copy(yhi_hbm.at[colsb.at[p]], g_v.at[p],
                                     sg[p])

            def wait_gather(p):
                pltpu.make_async_copy(ylo_hbm.at[colsb.at[p]], g_v.at[p],
                                      sg[p]).wait()

            def start_scat(p):
                pltpu.async_copy(g_v.at[p], acc.at[rows_s.at[p]], ss[p],
                                 add=True)

            def wait_scat(p):
                pltpu.make_async_copy(g_v.at[p], acc.at[rows_s.at[p]],
                                      ss[p]).wait()

            def scale(p):
                def scale_blk(ii, carry2):
                    for u in range(UNROLL):
                        i = ii * UNROLL + u
                        vv = vals_v[p, ii * (UNROLL // 8) + u // 8,
                                    pl.ds((u % 8) * 16, 16)]
                        for r in range(H // 16):
                            sl = pl.ds(r * 16, 16)
                            g_v[p, i, sl] = g_v[p, i, sl] * vv
                    return carry2

                lax.fori_loop(0, CHUNK // UNROLL, scale_blk, 0,
                              unroll=False)

            for p in range(NSLOT):
                start_meta(0, p)
            wait_meta(0)
            start_gather(0)

            def iter_body(t, carry):
                for u in range(NSLOT):
                    v = (u + 1) % NSLOT

                    def prefetch():
                        if u == NSLOT - 1:
                            wait_scat(v)
                        else:
                            @pl.when(t >= 1)
                            def _():
                                wait_scat(v)

                        wait_meta(v)
                        start_gather(v)

                    if u == NSLOT - 1:
                        @pl.when(t < NPAIR - 1)
                        def _():
                            prefetch()
                    else:
                        prefetch()

                    wait_gather(u)
                    scale(u)
                    for r in range(CHUNK // 16):
                        sl = pl.ds(r * 16, 16)
                        rows_s[u, sl] = rowsb[u, sl]
                    start_scat(u)

                    @pl.when(t < NPAIR - 1)
                    def _():
                        start_meta(t + 1, u)
                return carry

            lax.fori_loop(0, NPAIR, iter_body, 0)
            for p in range(NSLOT):
                wait_scat(p)

        edge_pass()

        plsc.subcore_barrier()

        def wb(col0):
            pltpu.sync_copy(
                acc.at[pl.ds(r0, ROWS_PER_TILE)],
                out_hbm.at[pl.ds(r0, ROWS_PER_TILE), pl.ds(col0, H)])

            @pl.when(s == 0)
            def _():
                tail = SUBC * ROWS_PER_TILE
                pltpu.sync_copy(
                    acc.at[pl.ds(tail, ROWS_REM)],
                    out_hbm.at[pl.ds(tail, ROWS_REM), pl.ds(col0, H)])

        @pl.when(c == 0)
        def _():
            wb(0)

        @pl.when(c == 1)
        def _():
            wb(H)

    return k(y_lo, y_hi, *cols3, *rows3, *vals3, bb_lo, bb_hi)


def _tc_matmul(x, w):
    BM = 1000

    def mm(x_ref, w_ref, ylo_ref, yhi_ref):
        y = jnp.dot(x_ref[...], w_ref[...],
                    preferred_element_type=jnp.float32)
        ylo_ref[...] = y[:, :H]
        yhi_ref[...] = y[:, H:]

    return pl.pallas_call(
        mm,
        grid=(N // BM,),
        in_specs=[
            pl.BlockSpec((BM, D), lambda m: (m, 0)),
            pl.BlockSpec((D, D), lambda m: (0, 0)),
        ],
        out_specs=(
            pl.BlockSpec((BM, H), lambda m: (m, 0)),
            pl.BlockSpec((BM, H), lambda m: (m, 0)),
        ),
        out_shape=(
            jax.ShapeDtypeStruct((N, H), jnp.float32),
            jax.ShapeDtypeStruct((N, H), jnp.float32),
        ),
    )(x, w)


@jax.jit
def kernel(x, F0_rows, F0_cols, F0_vals, F1_rows, F1_cols, F1_vals,
           F2_rows, F2_cols, F2_vals, W, b):
    y_lo, y_hi = _tc_matmul(x, W)
    vals3 = tuple(
        jnp.broadcast_to(v.reshape(NCH_M, CHUNK, 1),
                         (NCH_M, CHUNK, 16)).reshape(NCH_M, VROWS, 128)
        for v in (F0_vals, F1_vals, F2_vals))
    bb_lo = jnp.broadcast_to(b[None, :H], (ROWS_PER_TILE, H))
    bb_hi = jnp.broadcast_to(b[None, H:], (ROWS_PER_TILE, H))
    return _sc_spmm(y_lo, y_hi, (F0_cols, F1_cols, F2_cols),
                    (F0_rows, F1_rows, F2_rows), vals3, bb_lo, bb_hi)


# speedup vs baseline: 1.6566x; 1.3516x over previous
import functools

import jax
import jax.numpy as jnp
from jax import lax
from jax.experimental import pallas as pl
from jax.experimental.pallas import tpu as pltpu
from jax.experimental.pallas import tpu_sc as plsc

N = 10000
D = 256
H = 128
E = 160000
SUBC = 16
EPT_M = E // SUBC
CHUNK = 80
NPAIR = EPT_M // CHUNK
NCH_M = E // CHUNK
NSLOT = 3
ROWS_PER_TILE = 624
ROWS_REM = N - SUBC * ROWS_PER_TILE
UNROLL = 8
VROWS = CHUNK * 16 // 128


def _sc_spmm(y_lo, y_hi, cols3, rows3, vals3, bb_lo, bb_hi):
    mesh = plsc.VectorSubcoreMesh(core_axis_name="c", subcore_axis_name="s")

    @functools.partial(
        pl.kernel,
        mesh=mesh,
        out_type=jax.ShapeDtypeStruct((N, D), jnp.float32),
        scratch_types=[
            pltpu.VMEM((NSLOT, CHUNK), jnp.int32),
            pltpu.VMEM((NSLOT, CHUNK), jnp.int32),
            pltpu.VMEM((NSLOT, CHUNK), jnp.int32),
            pltpu.VMEM((NSLOT, CHUNK), jnp.float32),
            pltpu.VMEM((NSLOT, CHUNK, H), jnp.float32),
            pltpu.VMEM_SHARED((N, H), jnp.float32),
            [pltpu.SemaphoreType.DMA] * NSLOT,
            [pltpu.SemaphoreType.DMA] * NSLOT,
            [pltpu.SemaphoreType.DMA] * NSLOT,
        ],
    )
    def k(ylo_hbm, yhi_hbm, c0_hbm, c1_hbm, c2_hbm, r0_hbm, r1_hbm, r2_hbm,
          v0_hbm, v1_hbm, v2_hbm, blo_hbm, bhi_hbm, out_hbm,
          colsb, rowsb, rows_s, vals_v, g_v, acc, sg, sv, ss):
        c = lax.axis_index("c")
        s = lax.axis_index("s")
        r0 = s * ROWS_PER_TILE
        e0 = s * EPT_M
        c0 = s * NPAIR
        cols_h = (c0_hbm, c1_hbm, c2_hbm)
        rows_h = (r0_hbm, r1_hbm, r2_hbm)
        vals_h = (v0_hbm, v1_hbm, v2_hbm)

        def init_acc(b_hbm):
            pltpu.sync_copy(b_hbm, acc.at[pl.ds(r0, ROWS_PER_TILE)])

            @pl.when(s == 0)
            def _():
                pltpu.sync_copy(b_hbm.at[pl.ds(0, ROWS_REM)],
                                acc.at[pl.ds(SUBC * ROWS_PER_TILE,
                                             ROWS_REM)])

        @pl.when(c == 0)
        def _():
            init_acc(blo_hbm)

        @pl.when(c == 1)
        def _():
            init_acc(bhi_hbm)

        plsc.subcore_barrier()

        def edge_pass():
            def start_meta(tt, p):
                off = e0 + tt * CHUNK
                pltpu.async_copy(cols_h[p].at[pl.ds(off, CHUNK)],
                                 colsb.at[p], sv[p])
                pltpu.async_copy(rows_h[p].at[pl.ds(off, CHUNK)],
                                 rowsb.at[p], sv[p])
                pltpu.async_copy(vals_h[p].at[pl.ds(off, CHUNK)],
                                 vals_v.at[p], sv[p])

            def wait_meta(p):
                pltpu.make_async_copy(cols_h[p].at[pl.ds(e0, CHUNK)],
                                      colsb.at[p], sv[p]).wait()
                pltpu.make_async_copy(rows_h[p].at[pl.ds(e0, CHUNK)],
                                      rowsb.at[p], sv[p]).wait()
                pltpu.make_async_copy(vals_h[p].at[pl.ds(e0, CHUNK)],
                                      vals_v.at[p], sv[p]).wait()

            def start_gather(p):
                @pl.when(c == 0)
                def _():
                    pltpu.async_copy(ylo_hbm.at[colsb.at[p]], g_v.at[p],
                                     sg[p])

                @pl.when(c == 1)
                def _():
                    pltpu.async_copy(yhi_hbm.at[colsb.at[p]], g_v.at[p],
                                     sg[p])

            def wait_gather(p):
                pltpu.make_async_copy(ylo_hbm.at[colsb.at[p]], g_v.at[p],
                                      sg[p]).wait()

            def start_scat(p):
                pltpu.async_copy(g_v.at[p], acc.at[rows_s.at[p]], ss[p],
                                 add=True)

            def wait_scat(p):
                pltpu.make_async_copy(g_v.at[p], acc.at[rows_s.at[p]],
                                      ss[p]).wait()

            lane_ids = [jnp.full((16, 1), u, jnp.int32) for u in range(16)]
            dnums = lax.GatherDimensionNumbers(
                offset_dims=(), collapsed_slice_dims=(0,),
                start_index_map=(0,))

            def scale(p):
                def scale_blk(ii, carry2):
                    vrow = vals_v[p, pl.ds(ii * 16, 16)]
                    for u in range(16):
                        i = ii * 16 + u
                        vv = lax.gather(
                            vrow, lane_ids[u], dimension_numbers=dnums,
                            slice_sizes=(1,),
                            mode=lax.GatherScatterMode.PROMISE_IN_BOUNDS)
                        for r in range(H // 16):
                            sl = pl.ds(r * 16, 16)
                            g_v[p, i, sl] = g_v[p, i, sl] * vv
                    return carry2

                lax.fori_loop(0, CHUNK // 16, scale_blk, 0,
                              unroll=False)

            for p in range(NSLOT):
                start_meta(0, p)
            wait_meta(0)
            start_gather(0)

            def iter_body(t, carry):
                for u in range(NSLOT):
                    v = (u + 1) % NSLOT

                    def prefetch():
                        if u == NSLOT - 1:
                            wait_scat(v)
                        else:
                            @pl.when(t >= 1)
                            def _():
                                wait_scat(v)

                        wait_meta(v)
                        start_gather(v)

                    if u == NSLOT - 1:
                        @pl.when(t < NPAIR - 1)
                        def _():
                            prefetch()
                    else:
                        prefetch()

                    wait_gather(u)
                    scale(u)
                    for r in range(CHUNK // 16):
                        sl = pl.ds(r * 16, 16)
                        rows_s[u, sl] = rowsb[u, sl]
                    start_scat(u)

                    @pl.when(t < NPAIR - 1)
                    def _():
                        start_meta(t + 1, u)
                return carry

            lax.fori_loop(0, NPAIR, iter_body, 0)
            for p in range(NSLOT):
                wait_scat(p)

        edge_pass()

        plsc.subcore_barrier()

        def wb(col0):
            pltpu.sync_copy(
                acc.at[pl.ds(r0, ROWS_PER_TILE)],
                out_hbm.at[pl.ds(r0, ROWS_PER_TILE), pl.ds(col0, H)])

            @pl.when(s == 0)
            def _():
                tail = SUBC * ROWS_PER_TILE
                pltpu.sync_copy(
                    acc.at[pl.ds(tail, ROWS_REM)],
                    out_hbm.at[pl.ds(tail, ROWS_REM), pl.ds(col0, H)])

        @pl.when(c == 0)
        def _():
            wb(0)

        @pl.when(c == 1)
        def _():
            wb(H)

    return k(y_lo, y_hi, *cols3, *rows3, *vals3, bb_lo, bb_hi)


def _tc_matmul(x, w):
    BM = 1000

    def mm(x_ref, w_ref, ylo_ref, yhi_ref):
        y = jnp.dot(x_ref[...], w_ref[...],
                    preferred_element_type=jnp.float32)
        ylo_ref[...] = y[:, :H]
        yhi_ref[...] = y[:, H:]

    return pl.pallas_call(
        mm,
        grid=(N // BM,),
        in_specs=[
            pl.BlockSpec((BM, D), lambda m: (m, 0)),
            pl.BlockSpec((D, D), lambda m: (0, 0)),
        ],
        out_specs=(
            pl.BlockSpec((BM, H), lambda m: (m, 0)),
            pl.BlockSpec((BM, H), lambda m: (m, 0)),
        ),
        out_shape=(
            jax.ShapeDtypeStruct((N, H), jnp.float32),
            jax.ShapeDtypeStruct((N, H), jnp.float32),
        ),
    )(x, w)


@jax.jit
def kernel(x, F0_rows, F0_cols, F0_vals, F1_rows, F1_cols, F1_vals,
           F2_rows, F2_cols, F2_vals, W, b):
    y_lo, y_hi = _tc_matmul(x, W)
    vals3 = (F0_vals, F1_vals, F2_vals)
    bb_lo = jnp.broadcast_to(b[None, :H], (ROWS_PER_TILE, H))
    bb_hi = jnp.broadcast_to(b[None, H:], (ROWS_PER_TILE, H))
    return _sc_spmm(y_lo, y_hi, (F0_cols, F1_cols, F2_cols),
                    (F0_rows, F1_rows, F2_rows), vals3, bb_lo, bb_hi)


# speedup vs baseline: 1.6594x; 1.0017x over previous
import functools

import jax
import jax.numpy as jnp
from jax import lax
from jax.experimental import pallas as pl
from jax.experimental.pallas import tpu as pltpu
from jax.experimental.pallas import tpu_sc as plsc

N = 10000
D = 256
H = 128
E = 160000
SUBC = 16
EPT_M = E // SUBC
CHUNK = 80
NPAIR = EPT_M // CHUNK
NCH_M = E // CHUNK
NSLOT = 3
ROWS_PER_TILE = 624
ROWS_REM = N - SUBC * ROWS_PER_TILE
UNROLL = 8
VROWS = CHUNK * 16 // 128


def _sc_spmm(y_lo, y_hi, cols3, rows3, vals3, bb_lo, bb_hi):
    mesh = plsc.VectorSubcoreMesh(core_axis_name="c", subcore_axis_name="s")

    @functools.partial(
        pl.kernel,
        mesh=mesh,
        out_type=jax.ShapeDtypeStruct((N, D), jnp.float32),
        scratch_types=[
            pltpu.VMEM((NSLOT, CHUNK), jnp.int32),
            pltpu.VMEM((NSLOT, CHUNK), jnp.int32),
            pltpu.VMEM((NSLOT, CHUNK), jnp.int32),
            pltpu.VMEM((NSLOT, CHUNK), jnp.float32),
            pltpu.VMEM((NSLOT, CHUNK, H), jnp.float32),
            pltpu.VMEM_SHARED((N, H), jnp.float32),
            [pltpu.SemaphoreType.DMA] * NSLOT,
            [pltpu.SemaphoreType.DMA] * NSLOT,
            [pltpu.SemaphoreType.DMA] * NSLOT,
        ],
    )
    def k(ylo_hbm, yhi_hbm, c0_hbm, c1_hbm, c2_hbm, r0_hbm, r1_hbm, r2_hbm,
          v0_hbm, v1_hbm, v2_hbm, blo_hbm, bhi_hbm, out_hbm,
          colsb, rowsb, rows_s, vals_v, g_v, acc, sg, sv, ss):
        c = lax.axis_index("c")
        s = lax.axis_index("s")
        r0 = s * ROWS_PER_TILE
        e0 = s * EPT_M
        c0 = s * NPAIR
        cols_h = (c0_hbm, c1_hbm, c2_hbm)
        rows_h = (r0_hbm, r1_hbm, r2_hbm)
        vals_h = (v0_hbm, v1_hbm, v2_hbm)

        def init_acc(b_hbm):
            pltpu.sync_copy(b_hbm, acc.at[pl.ds(r0, ROWS_PER_TILE)])

            @pl.when(s == 0)
            def _():
                pltpu.sync_copy(b_hbm.at[pl.ds(0, ROWS_REM)],
                                acc.at[pl.ds(SUBC * ROWS_PER_TILE,
                                             ROWS_REM)])

        @pl.when(c == 0)
        def _():
            init_acc(blo_hbm)

        @pl.when(c == 1)
        def _():
            init_acc(bhi_hbm)

        plsc.subcore_barrier()

        def edge_pass():
            def start_meta(tt, p):
                off = e0 + tt * CHUNK
                pltpu.async_copy(cols_h[p].at[pl.ds(off, CHUNK)],
                                 colsb.at[p], sv[p])
                pltpu.async_copy(rows_h[p].at[pl.ds(off, CHUNK)],
                                 rowsb.at[p], sv[p])
                pltpu.async_copy(vals_h[p].at[pl.ds(off, CHUNK)],
                                 vals_v.at[p], sv[p])

            def wait_meta(p):
                pltpu.make_async_copy(cols_h[p].at[pl.ds(e0, CHUNK)],
                                      colsb.at[p], sv[p]).wait()
                pltpu.make_async_copy(rows_h[p].at[pl.ds(e0, CHUNK)],
                                      rowsb.at[p], sv[p]).wait()
                pltpu.make_async_copy(vals_h[p].at[pl.ds(e0, CHUNK)],
                                      vals_v.at[p], sv[p]).wait()

            def start_gather(p):
                @pl.when(c == 0)
                def _():
                    pltpu.async_copy(ylo_hbm.at[colsb.at[p]], g_v.at[p],
                                     sg[p])

                @pl.when(c == 1)
                def _():
                    pltpu.async_copy(yhi_hbm.at[colsb.at[p]], g_v.at[p],
                                     sg[p])

            def wait_gather(p):
                pltpu.make_async_copy(ylo_hbm.at[colsb.at[p]], g_v.at[p],
                                      sg[p]).wait()

            def start_scat(p):
                pltpu.async_copy(g_v.at[p], acc.at[rows_s.at[p]], ss[p],
                                 add=True)

            def wait_scat(p):
                pltpu.make_async_copy(g_v.at[p], acc.at[rows_s.at[p]],
                                      ss[p]).wait()

            lane_ids = [jnp.full((16, 1), u, jnp.int32) for u in range(16)]
            dnums = lax.GatherDimensionNumbers(
                offset_dims=(), collapsed_slice_dims=(0,),
                start_index_map=(0,))

            def scale(p):
                def scale_blk(ii, carry2):
                    vrow = vals_v[p, pl.ds(ii * 16, 16)]
                    for u in range(16):
                        i = ii * 16 + u
                        vv = lax.gather(
                            vrow, lane_ids[u], dimension_numbers=dnums,
                            slice_sizes=(1,),
                            mode=lax.GatherScatterMode.PROMISE_IN_BOUNDS)
                        for r in range(H // 16):
                            sl = pl.ds(r * 16, 16)
                            g_v[p, i, sl] = g_v[p, i, sl] * vv
                    return carry2

                lax.fori_loop(0, CHUNK // 16, scale_blk, 0,
                              unroll=False)

            for p in range(NSLOT):
                start_meta(0, p)
            wait_meta(0)
            start_gather(0)

            def iter_body(t, carry):
                for u in range(NSLOT):
                    v = (u + 1) % NSLOT

                    def prefetch():
                        if u == NSLOT - 1:
                            wait_scat(v)
                        else:
                            @pl.when(t >= 1)
                            def _():
                                wait_scat(v)

                        wait_meta(v)
                        start_gather(v)

                    if u == NSLOT - 1:
                        @pl.when(t < NPAIR - 1)
                        def _():
                            prefetch()
                    else:
                        prefetch()

                    for r in range(CHUNK // 16):
                        sl = pl.ds(r * 16, 16)
                        rows_s[u, sl] = rowsb[u, sl]
                    wait_gather(u)
                    scale(u)
                    start_scat(u)

                    @pl.when(t < NPAIR - 1)
                    def _():
                        start_meta(t + 1, u)
                return carry

            lax.fori_loop(0, NPAIR, iter_body, 0)
            for p in range(NSLOT):
                wait_scat(p)

        edge_pass()

        plsc.subcore_barrier()

        def wb(col0):
            pltpu.sync_copy(
                acc.at[pl.ds(r0, ROWS_PER_TILE)],
                out_hbm.at[pl.ds(r0, ROWS_PER_TILE), pl.ds(col0, H)])

            @pl.when(s == 0)
            def _():
                tail = SUBC * ROWS_PER_TILE
                pltpu.sync_copy(
                    acc.at[pl.ds(tail, ROWS_REM)],
                    out_hbm.at[pl.ds(tail, ROWS_REM), pl.ds(col0, H)])

        @pl.when(c == 0)
        def _():
            wb(0)

        @pl.when(c == 1)
        def _():
            wb(H)

    return k(y_lo, y_hi, *cols3, *rows3, *vals3, bb_lo, bb_hi)


def _tc_matmul(x, w):
    BM = 1000

    def mm(x_ref, w_ref, ylo_ref, yhi_ref):
        y = jnp.dot(x_ref[...], w_ref[...],
                    preferred_element_type=jnp.float32)
        ylo_ref[...] = y[:, :H]
        yhi_ref[...] = y[:, H:]

    return pl.pallas_call(
        mm,
        grid=(N // BM,),
        in_specs=[
            pl.BlockSpec((BM, D), lambda m: (m, 0)),
            pl.BlockSpec((D, D), lambda m: (0, 0)),
        ],
        out_specs=(
            pl.BlockSpec((BM, H), lambda m: (m, 0)),
            pl.BlockSpec((BM, H), lambda m: (m, 0)),
        ),
        out_shape=(
            jax.ShapeDtypeStruct((N, H), jnp.float32),
            jax.ShapeDtypeStruct((N, H), jnp.float32),
        ),
    )(x, w)


@jax.jit
def kernel(x, F0_rows, F0_cols, F0_vals, F1_rows, F1_cols, F1_vals,
           F2_rows, F2_cols, F2_vals, W, b):
    y_lo, y_hi = _tc_matmul(x, W)
    vals3 = (F0_vals, F1_vals, F2_vals)
    bb_lo = jnp.broadcast_to(b[None, :H], (ROWS_PER_TILE, H))
    bb_hi = jnp.broadcast_to(b[None, H:], (ROWS_PER_TILE, H))
    return _sc_spmm(y_lo, y_hi, (F0_cols, F1_cols, F2_cols),
                    (F0_rows, F1_rows, F2_rows), vals3, bb_lo, bb_hi)
